# baseline (speedup 1.0000x reference)
import functools

import jax
import jax.numpy as jnp
from jax import lax
from jax.experimental import pallas as pl
from jax.experimental.pallas import tpu as pltpu
from jax.experimental.pallas import tpu_sc as plsc

N = 10000
E = 320000
D = 128
H = 128
C = 2

NC = 2
NS = 16
NW = NC * NS
EPW = E // NW
K = 80
NCHUNK = EPW // K
NP = 10112
RPT = NP // NS
ZR = 8

BN = 5000
GRID = N // BN


def _sc_segsum_body(z_hbm, ei_hbm, out_hbm,
                    si_v, dst_v, rows_v, zbuf_v, acc_sh,
                    gsem, ssem, isem, sem_z):
    cid = lax.axis_index("c")
    sid = lax.axis_index("s")
    wid = cid * NS + sid

    base = wid * EPW

    def _schunk(c):
        return ei_hbm.at[pl.ds(base + c * K, K)]

    pltpu.sync_copy(_schunk(0), si_v.at[0])
    pltpu.sync_copy(_schunk(1), si_v.at[1])
    pltpu.async_copy(z_hbm.at[si_v.at[0]], rows_v.at[0], gsem.at[0])
    pltpu.async_copy(z_hbm.at[si_v.at[1]], rows_v.at[1], gsem.at[1])
    pltpu.async_copy(_schunk(2), si_v.at[2], isem.at[2])

    def _dfire(c, carry):
        pltpu.async_copy(ei_hbm.at[pl.ds(E + base + c * K, K)], dst_v.at[c],
                         sem_z)
        return carry

    def _ddrain(c, carry):
        pltpu.make_async_copy(ei_hbm.at[pl.ds(E + base, K)], dst_v.at[0],
                              sem_z).wait()
        return carry

    lax.fori_loop(0, NCHUNK, _dfire, 0)
    lax.fori_loop(0, NCHUNK, _ddrain, 0)

    def _zinit(t, carry):
        zbuf_v[t // 8, pl.ds((t % 8) * 16, 16)] = jnp.zeros((16,), jnp.float32)
        return carry

    lax.fori_loop(0, ZR * (H // 16), _zinit, 0)
    r0 = sid * RPT

    def _zfire(q, carry):
        pltpu.async_copy(zbuf_v, acc_sh.at[pl.ds(r0 + q * ZR, ZR)], sem_z)
        return carry

    def _zdrain(q, carry):
        pltpu.make_async_copy(zbuf_v, acc_sh.at[pl.ds(r0, ZR)], sem_z).wait()
        return carry

    lax.fori_loop(0, RPT // ZR, _zfire, 0)
    lax.fori_loop(0, RPT // ZR, _zdrain, 0)
    plsc.subcore_barrier()

    def _wait_gather(b):
        pltpu.make_async_copy(z_hbm.at[si_v.at[0]], rows_v.at[b], gsem.at[b]).wait()

    def _wait_scatter(b):
        pltpu.make_async_copy(rows_v.at[b], acc_sh.at[dst_v.at[0]], ssem.at[b]).wait()

    def _step(c, carry):
        b = c % 3
        b2 = (c + 2) % 3
        _wait_gather(b)
        pltpu.async_copy(rows_v.at[b], acc_sh.at[dst_v.at[c]], ssem.at[b],
                         add=True)

        @pl.when(c + 2 < NCHUNK)
        def _():
            pltpu.make_async_copy(_schunk(0), si_v.at[b2], isem.at[b2]).wait()

            @pl.when(c >= 1)
            def _():
                _wait_scatter(b2)

            pltpu.async_copy(z_hbm.at[si_v.at[b2]], rows_v.at[b2], gsem.at[b2])

        @pl.when(c + 3 < NCHUNK)
        def _():
            pltpu.async_copy(_schunk(c + 3), si_v.at[b], isem.at[b])

        return carry

    lax.fori_loop(0, NCHUNK, _step, 0)
    _wait_scatter((NCHUNK - 1) % 3)
    _wait_scatter((NCHUNK - 2) % 3)
    _wait_scatter((NCHUNK - 3) % 3)
    plsc.subcore_barrier()

    pltpu.sync_copy(acc_sh.at[pl.ds(r0, RPT)], out_hbm.at[cid, pl.ds(r0, RPT)])


@functools.cache
def _sc_segsum():
    return pl.kernel(
        _sc_segsum_body,
        out_type=jax.ShapeDtypeStruct((NC, NP, H), jnp.float32),
        mesh=plsc.VectorSubcoreMesh(core_axis_name="c", subcore_axis_name="s",
                                    num_cores=NC, num_subcores=NS),
        scratch_types=[
            pltpu.VMEM((3, K), jnp.int32),
            pltpu.VMEM((NCHUNK, K), jnp.int32),
            pltpu.VMEM((3, K, H), jnp.float32),
            pltpu.VMEM((ZR, H), jnp.float32),
            pltpu.VMEM_SHARED((NP, H), jnp.float32),
            pltpu.SemaphoreType.DMA((3,)),
            pltpu.SemaphoreType.DMA((3,)),
            pltpu.SemaphoreType.DMA((3,)),
            pltpu.SemaphoreType.DMA,
        ],
    )


def _proj_body(x_ref, w_ref, p_ref, cs_ref):
    i = pl.program_id(0)
    xb = x_ref[...]
    p_ref[...] = jnp.dot(xb, w_ref[...], preferred_element_type=jnp.float32)
    bs = jnp.broadcast_to(jnp.sum(xb, axis=0, keepdims=True), (8, H))

    @pl.when(i == 0)
    def _():
        cs_ref[...] = jnp.zeros((8, H), jnp.float32)

    cs_ref[...] += bs


def _comb_body(p_ref, parts_ref, cs_ref, wn_ref, wr_ref, b_ref, h_ref):
    neigh = jnp.dot(parts_ref[0, :N, :] + parts_ref[1, :N, :], wn_ref[...],
                    preferred_element_type=jnp.float32)
    read = jnp.dot(cs_ref[0:1, :], wr_ref[...], preferred_element_type=jnp.float32)
    h = p_ref[...] + neigh + read + b_ref[...]
    h_ref[...] = jnp.maximum(h, 0.0)


def _fin_body(p_ref, parts_ref, cs_ref, wn_ref, wr_ref, b_ref,
              wo_ref, bo_ref, o_ref):
    neigh = jnp.dot(parts_ref[0, :N, :] + parts_ref[1, :N, :], wn_ref[...],
                    preferred_element_type=jnp.float32)
    read = jnp.dot(cs_ref[0:1, :], wr_ref[...], preferred_element_type=jnp.float32)
    h = p_ref[...] + neigh + read + b_ref[...]
    h = jnp.maximum(h, 0.0)
    logits_t = lax.dot_general(wo_ref[...], h, (((0,), (1,)), ((), ())),
                               preferred_element_type=jnp.float32) + bo_ref[...]
    o_ref[...] = 1.0 / (1.0 + jnp.exp(-logits_t))


_row_spec = pl.BlockSpec((BN, H), lambda i: (i, 0))
_w_spec = pl.BlockSpec((H, H), lambda i: (0, 0))
_v_spec = pl.BlockSpec((1, H), lambda i: (0, 0))
_cs_spec = pl.BlockSpec((8, H), lambda i: (0, 0))
_part0_spec = pl.BlockSpec((1, BN, H), lambda i: (0, i, 0))
_part1_spec = pl.BlockSpec((1, BN, H), lambda i: (1, i, 0))

_proj = pl.pallas_call(
    _proj_body,
    grid=(GRID,),
    in_specs=[_row_spec, _w_spec],
    out_specs=[_row_spec, _cs_spec],
    out_shape=[
        jax.ShapeDtypeStruct((N, H), jnp.float32),
        jax.ShapeDtypeStruct((8, H), jnp.float32),
    ],
)

_comb = pl.pallas_call(
    _comb_body,
    out_shape=jax.ShapeDtypeStruct((N, H), jnp.float32),
)

_fin = pl.pallas_call(
    _fin_body,
    out_shape=jax.ShapeDtypeStruct((C, N), jnp.float32),
)


@jax.jit
def kernel(x, edge_index, W_self_0, W_neigh_0, W_read_0, b_0,
           W_self_1, W_neigh_1, W_read_1, b_1, W_out, b_out):
    ei = edge_index.reshape(2 * E)

    parts0 = _sc_segsum()(x, ei)
    p0, cs0 = _proj(x, W_self_0)
    h0 = _comb(p0, parts0, cs0, W_neigh_0, W_read_0, b_0.reshape(1, H))
    parts1 = _sc_segsum()(h0, ei)
    p1, cs1 = _proj(h0, W_self_1)
    out_t = _fin(p1, parts1, cs1, W_neigh_1, W_read_1,
                 b_1.reshape(1, H), W_out, b_out.reshape(C, 1))
    return out_t.T


# speedup vs baseline: 1.0082x; 1.0082x over previous
import functools

import jax
import jax.numpy as jnp
from jax import lax
from jax.experimental import pallas as pl
from jax.experimental.pallas import tpu as pltpu
from jax.experimental.pallas import tpu_sc as plsc

N = 10000
E = 320000
D = 128
H = 128
C = 2

NC = 2
NS = 16
NW = NC * NS
EPW = E // NW
K = 80
NCHUNK = EPW // K
NP = 10112
RPT = NP // NS
ZR = 8

BN = 5000
GRID = N // BN


def _sc_segsum_body(z_hbm, ei_hbm, out_hbm,
                    si_v, dst_v, rows_v, zbuf_v, acc_sh,
                    gsem, ssem, isem, sem_z):
    cid = lax.axis_index("c")
    sid = lax.axis_index("s")
    wid = cid * NS + sid

    base = wid * EPW

    def _schunk(c):
        return ei_hbm.at[pl.ds(base + c * K, K)]

    pltpu.sync_copy(_schunk(0), si_v.at[0])
    pltpu.sync_copy(_schunk(1), si_v.at[1])
    pltpu.async_copy(z_hbm.at[si_v.at[0]], rows_v.at[0], gsem.at[0])
    pltpu.async_copy(z_hbm.at[si_v.at[1]], rows_v.at[1], gsem.at[1])
    pltpu.async_copy(_schunk(2), si_v.at[2], isem.at[2])

    def _dfire(c, carry):
        pltpu.async_copy(ei_hbm.at[pl.ds(E + base + c * K, K)], dst_v.at[c],
                         sem_z)
        return carry

    def _ddrain(c, carry):
        pltpu.make_async_copy(ei_hbm.at[pl.ds(E + base, K)], dst_v.at[0],
                              sem_z).wait()
        return carry

    lax.fori_loop(0, NCHUNK, _dfire, 0)
    lax.fori_loop(0, NCHUNK, _ddrain, 0)

    def _zinit(t, carry):
        zbuf_v[t // 8, pl.ds((t % 8) * 16, 16)] = jnp.zeros((16,), jnp.float32)
        return carry

    lax.fori_loop(0, ZR * (H // 16), _zinit, 0)
    r0 = sid * RPT

    def _zfire(q, carry):
        pltpu.async_copy(zbuf_v, acc_sh.at[pl.ds(r0 + q * ZR, ZR)], sem_z)
        return carry

    def _zdrain(q, carry):
        pltpu.make_async_copy(zbuf_v, acc_sh.at[pl.ds(r0, ZR)], sem_z).wait()
        return carry

    lax.fori_loop(0, RPT // ZR, _zfire, 0)
    lax.fori_loop(0, RPT // ZR, _zdrain, 0)
    plsc.subcore_barrier()

    def _wait_gather(b):
        pltpu.make_async_copy(z_hbm.at[si_v.at[0]], rows_v.at[b], gsem.at[b]).wait()

    def _wait_scatter(b):
        pltpu.make_async_copy(rows_v.at[b], acc_sh.at[dst_v.at[0]], ssem.at[b]).wait()

    def _step(c, carry):
        b = c % 3
        b2 = (c + 2) % 3
        _wait_gather(b)
        pltpu.async_copy(rows_v.at[b], acc_sh.at[dst_v.at[c]], ssem.at[b],
                         add=True)

        @pl.when(c + 2 < NCHUNK)
        def _():
            pltpu.make_async_copy(_schunk(0), si_v.at[b2], isem.at[b2]).wait()

            @pl.when(c >= 1)
            def _():
                _wait_scatter(b2)

            pltpu.async_copy(z_hbm.at[si_v.at[b2]], rows_v.at[b2], gsem.at[b2])

        @pl.when(c + 3 < NCHUNK)
        def _():
            pltpu.async_copy(_schunk(c + 3), si_v.at[b], isem.at[b])

        return carry

    lax.fori_loop(0, NCHUNK, _step, 0)
    _wait_scatter((NCHUNK - 1) % 3)
    _wait_scatter((NCHUNK - 2) % 3)
    _wait_scatter((NCHUNK - 3) % 3)
    plsc.subcore_barrier()

    pltpu.sync_copy(acc_sh.at[pl.ds(r0, RPT)], out_hbm.at[cid, pl.ds(r0, RPT)])


@functools.cache
def _sc_segsum():
    return pl.kernel(
        _sc_segsum_body,
        out_type=jax.ShapeDtypeStruct((NC, NP, H), jnp.float32),
        mesh=plsc.VectorSubcoreMesh(core_axis_name="c", subcore_axis_name="s",
                                    num_cores=NC, num_subcores=NS),
        scratch_types=[
            pltpu.VMEM((3, K), jnp.int32),
            pltpu.VMEM((NCHUNK, K), jnp.int32),
            pltpu.VMEM((3, K, H), jnp.float32),
            pltpu.VMEM((ZR, H), jnp.float32),
            pltpu.VMEM_SHARED((NP, H), jnp.float32),
            pltpu.SemaphoreType.DMA((3,)),
            pltpu.SemaphoreType.DMA((3,)),
            pltpu.SemaphoreType.DMA((3,)),
            pltpu.SemaphoreType.DMA,
        ],
    )


def _proj_body(x_ref, w_ref, p_ref, cs_ref):
    i = pl.program_id(0)
    xb = x_ref[...]
    p_ref[...] = jnp.dot(xb, w_ref[...], preferred_element_type=jnp.float32)
    bs = jnp.broadcast_to(jnp.sum(xb, axis=0, keepdims=True), (8, H))

    @pl.when(i == 0)
    def _():
        cs_ref[...] = jnp.zeros((8, H), jnp.float32)

    cs_ref[...] += bs


def _comb_body(p_ref, pa_ref, pb_ref, cs_ref, wn_ref, wr_ref, b_ref, h_ref):
    neigh = jnp.dot(pa_ref[0] + pb_ref[0], wn_ref[...],
                    preferred_element_type=jnp.float32)
    read = jnp.dot(cs_ref[0:1, :], wr_ref[...], preferred_element_type=jnp.float32)
    h = p_ref[...] + neigh + read + b_ref[...]
    h_ref[...] = jnp.maximum(h, 0.0)


def _fin_body(p_ref, parts_ref, cs_ref, wn_ref, wr_ref, b_ref,
              wo_ref, bo_ref, o_ref):
    neigh = jnp.dot(parts_ref[0, :N, :] + parts_ref[1, :N, :], wn_ref[...],
                    preferred_element_type=jnp.float32)
    read = jnp.dot(cs_ref[0:1, :], wr_ref[...], preferred_element_type=jnp.float32)
    h = p_ref[...] + neigh + read + b_ref[...]
    h = jnp.maximum(h, 0.0)
    logits_t = lax.dot_general(wo_ref[...], h, (((0,), (1,)), ((), ())),
                               preferred_element_type=jnp.float32) + bo_ref[...]
    o_ref[...] = 1.0 / (1.0 + jnp.exp(-logits_t))


_row_spec = pl.BlockSpec((BN, H), lambda i: (i, 0))
_w_spec = pl.BlockSpec((H, H), lambda i: (0, 0))
_v_spec = pl.BlockSpec((1, H), lambda i: (0, 0))
_cs_spec = pl.BlockSpec((8, H), lambda i: (0, 0))
_part0_spec = pl.BlockSpec((1, BN, H), lambda i: (0, i, 0))
_part1_spec = pl.BlockSpec((1, BN, H), lambda i: (1, i, 0))

_proj = pl.pallas_call(
    _proj_body,
    grid=(GRID,),
    in_specs=[_row_spec, _w_spec],
    out_specs=[_row_spec, _cs_spec],
    out_shape=[
        jax.ShapeDtypeStruct((N, H), jnp.float32),
        jax.ShapeDtypeStruct((8, H), jnp.float32),
    ],
)

_comb = pl.pallas_call(
    _comb_body,
    grid=(GRID,),
    in_specs=[_row_spec, _part0_spec, _part1_spec, _cs_spec, _w_spec, _w_spec,
              _v_spec],
    out_specs=_row_spec,
    out_shape=jax.ShapeDtypeStruct((N, H), jnp.float32),
)

_fin = pl.pallas_call(
    _fin_body,
    out_shape=jax.ShapeDtypeStruct((C, N), jnp.float32),
)


@jax.jit
def kernel(x, edge_index, W_self_0, W_neigh_0, W_read_0, b_0,
           W_self_1, W_neigh_1, W_read_1, b_1, W_out, b_out):
    ei = edge_index.reshape(2 * E)

    parts0 = _sc_segsum()(x, ei)
    p0, cs0 = _proj(x, W_self_0)
    h0 = _comb(p0, parts0, parts0, cs0, W_neigh_0, W_read_0, b_0.reshape(1, H))
    parts1 = _sc_segsum()(h0, ei)
    p1, cs1 = _proj(h0, W_self_1)
    out_t = _fin(p1, parts1, cs1, W_neigh_1, W_read_1,
                 b_1.reshape(1, H), W_out, b_out.reshape(C, 1))
    return out_t.T
